# baseline (speedup 1.0000x reference)
import functools

import jax
import jax.numpy as jnp
from jax import lax
from jax.experimental import pallas as pl
from jax.experimental.pallas import tpu as pltpu
from jax.experimental.pallas import tpu_sc as plsc

B = 4096
F = 512
G = 16
T = 16


def _sc_gathers(users, pos_items, neg_items, beta16, gamma_users, gamma_items,
                theta_users):
    info = plsc.get_sparse_core_info()
    nw = info.num_cores * info.num_subcores
    bpw = B // nw
    nslice = bpw // 16

    mesh = plsc.VectorSubcoreMesh(core_axis_name="c", subcore_axis_name="s")

    @functools.partial(
        pl.kernel,
        mesh=mesh,
        compiler_params=pltpu.CompilerParams(use_tc_tiling_on_sc=False,
                                             needs_layout_passes=False),
        out_type=[
            jax.ShapeDtypeStruct((B, G), jnp.float32),
            jax.ShapeDtypeStruct((B, T), jnp.float32),
            jax.ShapeDtypeStruct((B, G), jnp.float32),
            jax.ShapeDtypeStruct((B, G), jnp.float32),
            jax.ShapeDtypeStruct((B,), jnp.float32),
            jax.ShapeDtypeStruct((B,), jnp.float32),
        ],
        scratch_types=[
            pltpu.VMEM((bpw,), jnp.int32),
            pltpu.VMEM((bpw,), jnp.int32),
            pltpu.VMEM((bpw,), jnp.int32),
            pltpu.VMEM((bpw,), jnp.int32),
            pltpu.VMEM((bpw,), jnp.int32),
            pltpu.VMEM((bpw, G), jnp.float32),
            pltpu.VMEM((bpw, T), jnp.float32),
            pltpu.VMEM((bpw, G), jnp.float32),
            pltpu.VMEM((bpw, G), jnp.float32),
            pltpu.VMEM((bpw, 16), jnp.float32),
            pltpu.VMEM((bpw, 16), jnp.float32),
            pltpu.VMEM((bpw,), jnp.float32),
            pltpu.VMEM((bpw,), jnp.float32),
            pltpu.SemaphoreType.DMA,
        ],
    )
    def k(users_h, pos_h, neg_h, beta_h, gu_h, gi_h, tu_h,
          ug_h, ut_h, gp_h, gn_h, bp_h, bn_h,
          idx_u, idx_p, idx_n, brp_i, brn_i, ug_v, ut_v, gp_v, gn_v,
          bpr_v, bnr_v, bp_v, bn_v, sem):
        wid = lax.axis_index("s") * info.num_cores + lax.axis_index("c")
        base = wid * bpw
        pltpu.sync_copy(users_h.at[pl.ds(base, bpw)], idx_u)
        pltpu.sync_copy(pos_h.at[pl.ds(base, bpw)], idx_p)
        pltpu.sync_copy(neg_h.at[pl.ds(base, bpw)], idx_n)
        for i in range(nslice):
            sl = pl.ds(i * 16, 16)
            brp_i[sl] = lax.shift_right_logical(idx_p[sl], 4)
            brn_i[sl] = lax.shift_right_logical(idx_n[sl], 4)
        cps = [
            pltpu.async_copy(gu_h.at[idx_u], ug_v, sem),
            pltpu.async_copy(tu_h.at[idx_u], ut_v, sem),
            pltpu.async_copy(gi_h.at[idx_p], gp_v, sem),
            pltpu.async_copy(gi_h.at[idx_n], gn_v, sem),
            pltpu.async_copy(beta_h.at[brp_i], bpr_v, sem),
            pltpu.async_copy(beta_h.at[brn_i], bnr_v, sem),
        ]
        for cp in cps:
            cp.wait()
        for i in range(nslice):
            sl = pl.ds(i * 16, 16)
            rows = lax.iota(jnp.int32, 16) + (i * 16)
            bp_v[sl] = plsc.load_gather(
                bpr_v, [rows, lax.bitwise_and(idx_p[sl], 15)])
            bn_v[sl] = plsc.load_gather(
                bnr_v, [rows, lax.bitwise_and(idx_n[sl], 15)])
        pltpu.sync_copy(ug_v, ug_h.at[pl.ds(base, bpw)])
        pltpu.sync_copy(ut_v, ut_h.at[pl.ds(base, bpw)])
        pltpu.sync_copy(gp_v, gp_h.at[pl.ds(base, bpw)])
        pltpu.sync_copy(gn_v, gn_h.at[pl.ds(base, bpw)])
        pltpu.sync_copy(bp_v, bp_h.at[pl.ds(base, bpw)])
        pltpu.sync_copy(bn_v, bn_h.at[pl.ds(base, bpw)])

    return k(users, pos_items, neg_items, beta16, gamma_users, gamma_items,
             theta_users)


_ST_BLK = 512


def _st_body(pos_ref, neg_ref, e_ref, bpr_ref, ug_ref, ut_ref, gp_ref, gn_ref,
             bp_ref, bn_ref, s_ref, t_ref):
    fd = pos_ref[...] - neg_ref[...]
    tid = jnp.dot(fd, e_ref[...], preferred_element_type=jnp.float32,
                  precision=lax.Precision.HIGHEST)
    tcol = jnp.dot(fd, bpr_ref[...], preferred_element_type=jnp.float32,
                   precision=lax.Precision.HIGHEST)
    s_col = ((bp_ref[...] - bn_ref[...])
             + jnp.sum(ug_ref[...] * (gp_ref[...] - gn_ref[...]), axis=1,
                       keepdims=True)
             + jnp.sum(ut_ref[...] * tid, axis=1, keepdims=True))
    s_ref[...] = s_col
    t_ref[...] = tcol


def _compute_s_t(pos_feat, neg_feat, e, bpr, ug, ut, gp, gn, bp, bn):
    grid = (B // _ST_BLK,)
    row_blk = lambda i: (i, 0)
    full = lambda i: (0, 0)
    return pl.pallas_call(
        _st_body,
        grid=grid,
        in_specs=[
            pl.BlockSpec((_ST_BLK, F), row_blk),
            pl.BlockSpec((_ST_BLK, F), row_blk),
            pl.BlockSpec((F, G), full),
            pl.BlockSpec((F, 1), full),
            pl.BlockSpec((_ST_BLK, G), row_blk),
            pl.BlockSpec((_ST_BLK, T), row_blk),
            pl.BlockSpec((_ST_BLK, G), row_blk),
            pl.BlockSpec((_ST_BLK, G), row_blk),
            pl.BlockSpec((_ST_BLK, 1), row_blk),
            pl.BlockSpec((_ST_BLK, 1), row_blk),
        ],
        out_specs=[
            pl.BlockSpec((_ST_BLK, 1), row_blk),
            pl.BlockSpec((_ST_BLK, 1), row_blk),
        ],
        out_shape=[
            jax.ShapeDtypeStruct((B, 1), jnp.float32),
            jax.ShapeDtypeStruct((B, 1), jnp.float32),
        ],
    )(pos_feat, neg_feat, e, bpr, ug, ut, gp, gn, bp, bn)


_X_ROWS = 256


def _xuij_body(t_ref, s_ref, out_ref):
    out_ref[...] = t_ref[...] + s_ref[...]


def _compute_xuij(t_col, s_row):
    grid = (B // _X_ROWS,)
    return pl.pallas_call(
        _xuij_body,
        grid=grid,
        in_specs=[
            pl.BlockSpec((_X_ROWS, 1), lambda i: (i, 0)),
            pl.BlockSpec((1, B), lambda i: (0, 0)),
        ],
        out_specs=pl.BlockSpec((_X_ROWS, B), lambda i: (i, 0)),
        out_shape=jax.ShapeDtypeStruct((B, B), jnp.float32),
    )(t_col, s_row)


def kernel(users, pos_items, neg_items, pos_items_features, neg_items_features,
           beta_items, gamma_users, gamma_items, theta_users, E, beta_prime):
    users = users.astype(jnp.int32)
    pos_items = pos_items.astype(jnp.int32)
    neg_items = neg_items.astype(jnp.int32)
    beta16 = beta_items.reshape(-1, 16)

    ug, ut, gp, gn, bp, bn = _sc_gathers(
        users, pos_items, neg_items, beta16, gamma_users, gamma_items,
        theta_users)

    s_col, t_col = _compute_s_t(pos_items_features, neg_items_features, E,
                                beta_prime, ug, ut, gp, gn,
                                bp.reshape(B, 1), bn.reshape(B, 1))

    xuij = _compute_xuij(t_col, s_col.reshape(1, B))

    return (xuij, (ug, ut), (bp, bn), (gp, gn))


# speedup vs baseline: 7.8328x; 7.8328x over previous
import functools

import jax
import jax.numpy as jnp
from jax import lax
from jax.experimental import pallas as pl
from jax.experimental.pallas import tpu as pltpu
from jax.experimental.pallas import tpu_sc as plsc

B = 4096
F = 512
G = 16
T = 16


_NBANK = 4


def _sc_gathers(users, pos_items, neg_items, beta1d, gut, git, tut):
    info = plsc.get_sparse_core_info()
    nw = info.num_cores * info.num_subcores
    bpw = B // nw

    mesh = plsc.VectorSubcoreMesh(core_axis_name="c", subcore_axis_name="s")

    @functools.partial(
        pl.kernel,
        mesh=mesh,
        compiler_params=pltpu.CompilerParams(use_tc_tiling_on_sc=True,
                                             needs_layout_passes=False),
        out_type=[
            jax.ShapeDtypeStruct((G, B), jnp.float32),
            jax.ShapeDtypeStruct((T, B), jnp.float32),
            jax.ShapeDtypeStruct((G, B), jnp.float32),
            jax.ShapeDtypeStruct((G, B), jnp.float32),
            jax.ShapeDtypeStruct((B,), jnp.float32),
            jax.ShapeDtypeStruct((B,), jnp.float32),
        ],
        scratch_types=[
            pltpu.VMEM((bpw,), jnp.int32),
            pltpu.VMEM((bpw,), jnp.int32),
            pltpu.VMEM((bpw,), jnp.int32),
            [pltpu.VMEM((16, 128), jnp.float32) for _ in range(_NBANK)],
            pltpu.VMEM((G, bpw), jnp.float32),
            pltpu.VMEM((T, bpw), jnp.float32),
            pltpu.VMEM((G, bpw), jnp.float32),
            pltpu.VMEM((G, bpw), jnp.float32),
            pltpu.VMEM((bpw,), jnp.float32),
            pltpu.VMEM((bpw,), jnp.float32),
            pltpu.SemaphoreType.DMA,
            pltpu.SemaphoreType.DMA,
            pltpu.SemaphoreType.DMA,
            pltpu.SemaphoreType.DMA,
            pltpu.SemaphoreType.DMA,
        ],
    )
    def k(users_h, pos_h, neg_h, beta_h, gu_h, gi_h, tu_h,
          ugo, uto, gpo, gno, bpo, bno,
          iu_v, ip_v, in_v,
          banks, ug_v, ut_v, gp_v, gn_v, bp_v, bn_v,
          sem0, sem1, sem2, sem3, semb):
        sems = [sem0, sem1, sem2, sem3]
        wid = lax.axis_index("s") * info.num_cores + lax.axis_index("c")
        base = wid * bpw
        pltpu.sync_copy(users_h.at[pl.ds(base, bpw)], iu_v)
        pltpu.sync_copy(pos_h.at[pl.ds(base, bpw)], ip_v)
        pltpu.sync_copy(neg_h.at[pl.ds(base, bpw)], in_v)
        cpb1 = pltpu.async_copy(beta_h.at[ip_v], bp_v, semb)
        cpb2 = pltpu.async_copy(beta_h.at[in_v], bn_v, semb)
        lanes = lax.iota(jnp.int32, 16)
        zeros = jnp.full((16,), 0, jnp.int32)

        def bcast_idx(idx_ref, i):
            return plsc.load_gather(idx_ref, [zeros + i])

        def fetch(tab, idx_ref, i, p):
            bc = bcast_idx(idx_ref, i)
            col0_v = lax.shift_left(lax.shift_right_logical(bc, 7), 7)
            col0 = pl.multiple_of(lax.reduce_max(col0_v, (0,)), 128)
            pltpu.async_copy(tab.at[:, pl.ds(col0, 128)], banks[p], sems[p])

        def drain(tab, p):
            pltpu.make_async_copy(tab.at[:, pl.ds(0, 128)], banks[p],
                                  sems[p]).wait()

        def extract(idx_ref, i, p, out_v):
            col = lax.bitwise_and(bcast_idx(idx_ref, i), 127)
            val = plsc.load_gather(banks[p], [lanes, col])
            plsc.store_scatter(out_v, [lanes, zeros + i], val)

        for tab, idx_ref, out_v in ((gu_h, iu_v, ug_v), (tu_h, iu_v, ut_v),
                                    (gi_h, ip_v, gp_v), (gi_h, in_v, gn_v)):

            def body(b, _, tab=tab, idx_ref=idx_ref, out_v=out_v):
                for p in range(_NBANK):

                    @pl.when(b > 0)
                    def _():
                        drain(tab, p)
                        extract(idx_ref, (b - 1) * _NBANK + p, p, out_v)

                    @pl.when(b * _NBANK + p < bpw)
                    def _():
                        fetch(tab, idx_ref, b * _NBANK + p, p)
                return 0

            lax.fori_loop(0, bpw // _NBANK + 1, body, 0)
        cpb1.wait()
        cpb2.wait()
        csl = pl.ds(base, bpw)
        pltpu.sync_copy(ug_v, ugo.at[:, csl])
        pltpu.sync_copy(ut_v, uto.at[:, csl])
        pltpu.sync_copy(gp_v, gpo.at[:, csl])
        pltpu.sync_copy(gn_v, gno.at[:, csl])
        pltpu.sync_copy(bp_v, bpo.at[csl])
        pltpu.sync_copy(bn_v, bno.at[csl])

    return k(users, pos_items, neg_items, beta1d, gut, git, tut)


_T1_BLK = 512


def _t1_body(pos_ref, neg_ref, et_ref, bpr_ref, tid_ref, t_ref):
    fd = pos_ref[...] - neg_ref[...]
    tid_ref[...] = lax.dot_general(
        et_ref[...], fd, (((1,), (1,)), ((), ())),
        preferred_element_type=jnp.float32,
        precision=lax.Precision.HIGHEST)
    t_ref[...] = jnp.dot(fd, bpr_ref[...], preferred_element_type=jnp.float32,
                         precision=lax.Precision.HIGHEST)


def _compute_t1(pos_feat, neg_feat, e_t, bpr):
    grid = (B // _T1_BLK,)
    return pl.pallas_call(
        _t1_body,
        grid=grid,
        in_specs=[
            pl.BlockSpec((_T1_BLK, F), lambda i: (i, 0)),
            pl.BlockSpec((_T1_BLK, F), lambda i: (i, 0)),
            pl.BlockSpec((T, F), lambda i: (0, 0)),
            pl.BlockSpec((F, 1), lambda i: (0, 0)),
        ],
        out_specs=[
            pl.BlockSpec((T, _T1_BLK), lambda i: (0, i)),
            pl.BlockSpec((_T1_BLK, 1), lambda i: (i, 0)),
        ],
        out_shape=[
            jax.ShapeDtypeStruct((T, B), jnp.float32),
            jax.ShapeDtypeStruct((B, 1), jnp.float32),
        ],
    )(pos_feat, neg_feat, e_t, bpr)


def _t2_body(ug_ref, ut_ref, gp_ref, gn_ref, tid_ref, bp_ref, bn_ref, s_ref):
    s = (bp_ref[...] - bn_ref[...]
         + jnp.sum(ug_ref[...] * (gp_ref[...] - gn_ref[...]), axis=0,
                   keepdims=True)
         + jnp.sum(ut_ref[...] * tid_ref[...], axis=0, keepdims=True))
    s_ref[...] = s


def _compute_s(ug_t, ut_t, gp_t, gn_t, tid_t, bp_row, bn_row):
    return pl.pallas_call(
        _t2_body,
        out_shape=jax.ShapeDtypeStruct((1, B), jnp.float32),
    )(ug_t, ut_t, gp_t, gn_t, tid_t, bp_row, bn_row)


_X_ROWS = 256


def _xuij_body(t_ref, s_ref, out_ref):
    out_ref[...] = t_ref[...] + s_ref[...]


def _compute_xuij(t_col, s_row):
    grid = (B // _X_ROWS,)
    return pl.pallas_call(
        _xuij_body,
        grid=grid,
        in_specs=[
            pl.BlockSpec((_X_ROWS, 1), lambda i: (i, 0)),
            pl.BlockSpec((1, B), lambda i: (0, 0)),
        ],
        out_specs=pl.BlockSpec((_X_ROWS, B), lambda i: (i, 0)),
        out_shape=jax.ShapeDtypeStruct((B, B), jnp.float32),
    )(t_col, s_row)


def kernel(users, pos_items, neg_items, pos_items_features, neg_items_features,
           beta_items, gamma_users, gamma_items, theta_users, E, beta_prime):
    users = users.astype(jnp.int32)
    pos_items = pos_items.astype(jnp.int32)
    neg_items = neg_items.astype(jnp.int32)

    gut = jnp.transpose(gamma_users)
    git = jnp.transpose(gamma_items)
    tut = jnp.transpose(theta_users)

    ug_t, ut_t, gp_t, gn_t, bp, bn = _sc_gathers(
        users, pos_items, neg_items, beta_items, gut, git, tut)

    tid_t, t_col = _compute_t1(pos_items_features, neg_items_features,
                               jnp.transpose(E), beta_prime)

    s_row = _compute_s(ug_t, ut_t, gp_t, gn_t, tid_t,
                       bp.reshape(1, B), bn.reshape(1, B))

    xuij = _compute_xuij(t_col, s_row)

    return (xuij,
            (jnp.transpose(ug_t), jnp.transpose(ut_t)),
            (bp, bn),
            (jnp.transpose(gp_t), jnp.transpose(gn_t)))


# speedup vs baseline: 9.9208x; 1.2666x over previous
import functools

import jax
import jax.numpy as jnp
from jax import lax
from jax.experimental import pallas as pl
from jax.experimental.pallas import tpu as pltpu
from jax.experimental.pallas import tpu_sc as plsc

B = 4096
F = 512
G = 16
T = 16


_NBANK = 4


def _sc_gathers(users, pos_items, neg_items, beta1d, gut, git, tut):
    info = plsc.get_sparse_core_info()
    nw = info.num_cores * info.num_subcores
    bpw = B // nw

    mesh = plsc.VectorSubcoreMesh(core_axis_name="c", subcore_axis_name="s")

    @functools.partial(
        pl.kernel,
        mesh=mesh,
        compiler_params=pltpu.CompilerParams(use_tc_tiling_on_sc=True,
                                             needs_layout_passes=False),
        out_type=[
            jax.ShapeDtypeStruct((G, B), jnp.float32),
            jax.ShapeDtypeStruct((T, B), jnp.float32),
            jax.ShapeDtypeStruct((G, B), jnp.float32),
            jax.ShapeDtypeStruct((G, B), jnp.float32),
            jax.ShapeDtypeStruct((B,), jnp.float32),
            jax.ShapeDtypeStruct((B,), jnp.float32),
        ],
        scratch_types=[
            pltpu.VMEM((bpw,), jnp.int32),
            pltpu.VMEM((bpw,), jnp.int32),
            pltpu.VMEM((bpw,), jnp.int32),
            [[pltpu.VMEM((16, 128), jnp.float32) for _ in range(_NBANK)]
             for _ in range(4)],
            pltpu.VMEM((G, bpw), jnp.float32),
            pltpu.VMEM((T, bpw), jnp.float32),
            pltpu.VMEM((G, bpw), jnp.float32),
            pltpu.VMEM((G, bpw), jnp.float32),
            pltpu.VMEM((bpw,), jnp.float32),
            pltpu.VMEM((bpw,), jnp.float32),
            [[pltpu.SemaphoreType.DMA for _ in range(_NBANK)]
             for _ in range(4)],
            pltpu.SemaphoreType.DMA,
        ],
    )
    def k(users_h, pos_h, neg_h, beta_h, gu_h, gi_h, tu_h,
          ugo, uto, gpo, gno, bpo, bno,
          iu_v, ip_v, in_v,
          banks, ug_v, ut_v, gp_v, gn_v, bp_v, bn_v,
          sems, semb):
        wid = lax.axis_index("s") * info.num_cores + lax.axis_index("c")
        base = wid * bpw
        pltpu.sync_copy(users_h.at[pl.ds(base, bpw)], iu_v)
        pltpu.sync_copy(pos_h.at[pl.ds(base, bpw)], ip_v)
        pltpu.sync_copy(neg_h.at[pl.ds(base, bpw)], in_v)
        cpb1 = pltpu.async_copy(beta_h.at[ip_v], bp_v, semb)
        cpb2 = pltpu.async_copy(beta_h.at[in_v], bn_v, semb)
        lanes = lax.iota(jnp.int32, 16)
        zeros = jnp.full((16,), 0, jnp.int32)

        def bcast_idx(idx_ref, i):
            return plsc.load_gather(idx_ref, [zeros + i])

        def fetch(tab, idx_ref, i, s, p):
            bc = bcast_idx(idx_ref, i)
            col0_v = lax.shift_left(lax.shift_right_logical(bc, 7), 7)
            col0 = pl.multiple_of(lax.reduce_max(col0_v, (0,)), 128)
            pltpu.async_copy(tab.at[:, pl.ds(col0, 128)], banks[s][p],
                             sems[s][p])

        def drain(tab, s, p):
            pltpu.make_async_copy(tab.at[:, pl.ds(0, 128)], banks[s][p],
                                  sems[s][p]).wait()

        def extract(idx_ref, i, s, p, out_v):
            col = lax.bitwise_and(bcast_idx(idx_ref, i), 127)
            val = plsc.load_gather(banks[s][p], [lanes, col])
            plsc.store_scatter(out_v, [lanes, zeros + i], val)

        streams = ((gu_h, iu_v, ug_v), (tu_h, iu_v, ut_v),
                   (gi_h, ip_v, gp_v), (gi_h, in_v, gn_v))

        def body(b, _):
            for p in range(_NBANK):
                for s, (tab, idx_ref, out_v) in enumerate(streams):

                    @pl.when(b > 0)
                    def _(tab=tab, idx_ref=idx_ref, out_v=out_v, s=s, p=p):
                        drain(tab, s, p)
                        extract(idx_ref, (b - 1) * _NBANK + p, s, p, out_v)

                    @pl.when(b * _NBANK + p < bpw)
                    def _(tab=tab, idx_ref=idx_ref, s=s, p=p):
                        fetch(tab, idx_ref, b * _NBANK + p, s, p)

            return 0

        lax.fori_loop(0, bpw // _NBANK + 1, body, 0)
        cpb1.wait()
        cpb2.wait()
        csl = pl.ds(base, bpw)
        pltpu.sync_copy(ug_v, ugo.at[:, csl])
        pltpu.sync_copy(ut_v, uto.at[:, csl])
        pltpu.sync_copy(gp_v, gpo.at[:, csl])
        pltpu.sync_copy(gn_v, gno.at[:, csl])
        pltpu.sync_copy(bp_v, bpo.at[csl])
        pltpu.sync_copy(bn_v, bno.at[csl])

    return k(users, pos_items, neg_items, beta1d, gut, git, tut)


_T1_BLK = 512


def _t1_body(pos_ref, neg_ref, et_ref, bpr_ref, tid_ref, t_ref):
    fd = pos_ref[...] - neg_ref[...]
    tid_ref[...] = lax.dot_general(
        et_ref[...], fd, (((1,), (1,)), ((), ())),
        preferred_element_type=jnp.float32,
        precision=lax.Precision.HIGHEST)
    t_ref[...] = jnp.dot(fd, bpr_ref[...], preferred_element_type=jnp.float32,
                         precision=lax.Precision.HIGHEST)


def _compute_t1(pos_feat, neg_feat, e_t, bpr):
    grid = (B // _T1_BLK,)
    return pl.pallas_call(
        _t1_body,
        grid=grid,
        in_specs=[
            pl.BlockSpec((_T1_BLK, F), lambda i: (i, 0)),
            pl.BlockSpec((_T1_BLK, F), lambda i: (i, 0)),
            pl.BlockSpec((T, F), lambda i: (0, 0)),
            pl.BlockSpec((F, 1), lambda i: (0, 0)),
        ],
        out_specs=[
            pl.BlockSpec((T, _T1_BLK), lambda i: (0, i)),
            pl.BlockSpec((_T1_BLK, 1), lambda i: (i, 0)),
        ],
        out_shape=[
            jax.ShapeDtypeStruct((T, B), jnp.float32),
            jax.ShapeDtypeStruct((B, 1), jnp.float32),
        ],
    )(pos_feat, neg_feat, e_t, bpr)


def _t2_body(ug_ref, ut_ref, gp_ref, gn_ref, tid_ref, bp_ref, bn_ref, s_ref):
    s = (bp_ref[...] - bn_ref[...]
         + jnp.sum(ug_ref[...] * (gp_ref[...] - gn_ref[...]), axis=0,
                   keepdims=True)
         + jnp.sum(ut_ref[...] * tid_ref[...], axis=0, keepdims=True))
    s_ref[...] = s


def _compute_s(ug_t, ut_t, gp_t, gn_t, tid_t, bp_row, bn_row):
    return pl.pallas_call(
        _t2_body,
        out_shape=jax.ShapeDtypeStruct((1, B), jnp.float32),
    )(ug_t, ut_t, gp_t, gn_t, tid_t, bp_row, bn_row)


_X_ROWS = 256


def _xuij_body(t_ref, s_ref, out_ref):
    out_ref[...] = t_ref[...] + s_ref[...]


def _compute_xuij(t_col, s_row):
    grid = (B // _X_ROWS,)
    return pl.pallas_call(
        _xuij_body,
        grid=grid,
        in_specs=[
            pl.BlockSpec((_X_ROWS, 1), lambda i: (i, 0)),
            pl.BlockSpec((1, B), lambda i: (0, 0)),
        ],
        out_specs=pl.BlockSpec((_X_ROWS, B), lambda i: (i, 0)),
        out_shape=jax.ShapeDtypeStruct((B, B), jnp.float32),
    )(t_col, s_row)


def kernel(users, pos_items, neg_items, pos_items_features, neg_items_features,
           beta_items, gamma_users, gamma_items, theta_users, E, beta_prime):
    users = users.astype(jnp.int32)
    pos_items = pos_items.astype(jnp.int32)
    neg_items = neg_items.astype(jnp.int32)

    gut = jnp.transpose(gamma_users)
    git = jnp.transpose(gamma_items)
    tut = jnp.transpose(theta_users)

    ug_t, ut_t, gp_t, gn_t, bp, bn = _sc_gathers(
        users, pos_items, neg_items, beta_items, gut, git, tut)

    tid_t, t_col = _compute_t1(pos_items_features, neg_items_features,
                               jnp.transpose(E), beta_prime)

    s_row = _compute_s(ug_t, ut_t, gp_t, gn_t, tid_t,
                       bp.reshape(1, B), bn.reshape(1, B))

    xuij = _compute_xuij(t_col, s_row)

    return (xuij,
            (jnp.transpose(ug_t), jnp.transpose(ut_t)),
            (bp, bn),
            (jnp.transpose(gp_t), jnp.transpose(gn_t)))
